# Initial kernel scaffold; baseline (speedup 1.0000x reference)
import functools

import jax
import jax.numpy as jnp
from jax import lax
from jax.experimental import pallas as pl
from jax.experimental.pallas import tpu as pltpu
from jax.experimental.pallas import tpu_sc as plsc

_NC = 2
_NS = 16
_CHUNK = 80


def _sc_degree(dst_i32, n_nodes):
    e = dst_i32.shape[0]
    nw = _NC * _NS
    assert e % (nw * _CHUNK) == 0, e
    per_tile = e // nw
    chunks = per_tile // _CHUNK
    rpt = n_nodes // _NS
    assert n_nodes % _NS == 0

    mesh = plsc.VectorSubcoreMesh(core_axis_name="c", subcore_axis_name="s")

    @functools.partial(
        pl.kernel,
        out_type=jax.ShapeDtypeStruct((_NC, n_nodes, 8), jnp.float32),
        mesh=mesh,
        scratch_types=[
            pltpu.VMEM((_CHUNK,), jnp.int32),
            pltpu.VMEM((_CHUNK, 8), jnp.float32),
            pltpu.VMEM_SHARED((n_nodes, 8), jnp.float32),
        ],
    )
    def deg_kernel(dst_hbm, ones_h, zeros_h, out_hbm, didx_v, ones_v, acc_sh):
        cid = lax.axis_index("c")
        sid = lax.axis_index("s")
        pltpu.sync_copy(zeros_h.at[pl.ds(sid * rpt, rpt)],
                        acc_sh.at[pl.ds(sid * rpt, rpt)])
        pltpu.sync_copy(ones_h, ones_v)
        plsc.subcore_barrier()
        base0 = (cid * _NS + sid) * per_tile

        def body(j, carry):
            pltpu.sync_copy(dst_hbm.at[pl.ds(base0 + j * _CHUNK, _CHUNK)],
                            didx_v)
            pltpu.sync_copy(ones_v, acc_sh.at[didx_v], add=True)
            return carry

        lax.fori_loop(0, chunks, body, 0)
        plsc.subcore_barrier()
        pltpu.sync_copy(acc_sh.at[pl.ds(sid * rpt, rpt)],
                        out_hbm.at[cid, pl.ds(sid * rpt, rpt)])

    ones = jnp.ones((_CHUNK, 8), jnp.float32)
    zeros = jnp.zeros((n_nodes, 8), jnp.float32)
    return deg_kernel(dst_i32, ones, zeros)


def _sc_propagate(table, src_i32, dst_i32):
    n_nodes, d = table.shape
    e = src_i32.shape[0]
    nw = _NC * _NS
    assert e % (nw * _CHUNK) == 0, e
    per_tile = e // nw
    chunks = per_tile // _CHUNK
    rpt = n_nodes // _NS

    mesh = plsc.VectorSubcoreMesh(core_axis_name="c", subcore_axis_name="s")

    @functools.partial(
        pl.kernel,
        out_type=jax.ShapeDtypeStruct((_NC, n_nodes, d), jnp.float32),
        mesh=mesh,
        scratch_types=[
            pltpu.VMEM((_CHUNK,), jnp.int32),
            pltpu.VMEM((_CHUNK,), jnp.int32),
            pltpu.VMEM((_CHUNK, d), jnp.float32),
            pltpu.VMEM_SHARED((n_nodes, d), jnp.float32),
            pltpu.SemaphoreType.DMA,
        ],
    )
    def prop_kernel(table_hbm, src_hbm, dst_hbm, zeros_h, out_hbm,
                    sidx_v, didx_v, rows_v, acc_sh, sem):
        cid = lax.axis_index("c")
        sid = lax.axis_index("s")
        pltpu.sync_copy(zeros_h.at[pl.ds(sid * rpt, rpt)],
                        acc_sh.at[pl.ds(sid * rpt, rpt)])
        plsc.subcore_barrier()
        base0 = (cid * _NS + sid) * per_tile

        def body(j, carry):
            base = base0 + j * _CHUNK
            pltpu.sync_copy(src_hbm.at[pl.ds(base, _CHUNK)], sidx_v)
            pltpu.sync_copy(dst_hbm.at[pl.ds(base, _CHUNK)], didx_v)
            pltpu.async_copy(table_hbm.at[sidx_v], rows_v, sem).wait()
            pltpu.sync_copy(rows_v, acc_sh.at[didx_v], add=True)
            return carry

        lax.fori_loop(0, chunks, body, 0)
        plsc.subcore_barrier()
        pltpu.sync_copy(acc_sh.at[pl.ds(sid * rpt, rpt)],
                        out_hbm.at[cid, pl.ds(sid * rpt, rpt)])

    zeros = jnp.zeros((n_nodes, d), jnp.float32)
    return prop_kernel(table, src_i32, dst_i32, zeros)


_ROWS = 1000


def _tc_pre(x, deg0, deg1, w):
    n, d = x.shape
    dout = w.shape[1]
    assert n % _ROWS == 0

    def body(x_ref, d0_ref, d1_ref, w_ref, o_ref):
        dinv = lax.rsqrt(d0_ref[:, 0:1] + d1_ref[:, 0:1] + 1.0)
        o_ref[...] = jnp.dot(x_ref[...] * dinv, w_ref[...],
                             preferred_element_type=jnp.float32)

    return pl.pallas_call(
        body,
        grid=(n // _ROWS,),
        in_specs=[
            pl.BlockSpec((_ROWS, d), lambda i: (i, 0)),
            pl.BlockSpec((_ROWS, 8), lambda i: (i, 0)),
            pl.BlockSpec((_ROWS, 8), lambda i: (i, 0)),
            pl.BlockSpec((d, dout), lambda i: (0, 0)),
        ],
        out_specs=pl.BlockSpec((_ROWS, dout), lambda i: (i, 0)),
        out_shape=jax.ShapeDtypeStruct((n, dout), jnp.float32),
    )(x, deg0, deg1, w)


def _tc_mid(p0, p1, z, deg0, deg1, b, w):
    n, d = z.shape
    dout = w.shape[1]

    def body(p0_ref, p1_ref, z_ref, d0_ref, d1_ref, b_ref, w_ref, o_ref):
        dinv = lax.rsqrt(d0_ref[:, 0:1] + d1_ref[:, 0:1] + 1.0)
        pre = (p0_ref[...] + p1_ref[...] + z_ref[...]) * dinv + b_ref[...]
        h = jnp.maximum(pre, 0.0) * dinv
        o_ref[...] = jnp.dot(h, w_ref[...], preferred_element_type=jnp.float32)

    return pl.pallas_call(
        body,
        grid=(n // _ROWS,),
        in_specs=[
            pl.BlockSpec((_ROWS, d), lambda i: (i, 0)),
            pl.BlockSpec((_ROWS, d), lambda i: (i, 0)),
            pl.BlockSpec((_ROWS, d), lambda i: (i, 0)),
            pl.BlockSpec((_ROWS, 8), lambda i: (i, 0)),
            pl.BlockSpec((_ROWS, 8), lambda i: (i, 0)),
            pl.BlockSpec((1, d), lambda i: (0, 0)),
            pl.BlockSpec((d, dout), lambda i: (0, 0)),
        ],
        out_specs=pl.BlockSpec((_ROWS, dout), lambda i: (i, 0)),
        out_shape=jax.ShapeDtypeStruct((n, dout), jnp.float32),
    )(p0, p1, z, deg0, deg1, b, w)


def _tc_post(p0, p1, z, deg0, deg1, b, wh, bh):
    n, d = z.shape
    dout = wh.shape[1]

    def body(p0_ref, p1_ref, z_ref, d0_ref, d1_ref, b_ref, wh_ref, bh_ref,
             o_ref):
        dinv = lax.rsqrt(d0_ref[:, 0:1] + d1_ref[:, 0:1] + 1.0)
        pre = (p0_ref[...] + p1_ref[...] + z_ref[...]) * dinv + b_ref[...]
        h = jnp.maximum(pre, 0.0)
        o_ref[...] = jnp.dot(h, wh_ref[...],
                             preferred_element_type=jnp.float32) + bh_ref[...]

    return pl.pallas_call(
        body,
        grid=(n // _ROWS,),
        in_specs=[
            pl.BlockSpec((_ROWS, d), lambda i: (i, 0)),
            pl.BlockSpec((_ROWS, d), lambda i: (i, 0)),
            pl.BlockSpec((_ROWS, d), lambda i: (i, 0)),
            pl.BlockSpec((_ROWS, 8), lambda i: (i, 0)),
            pl.BlockSpec((_ROWS, 8), lambda i: (i, 0)),
            pl.BlockSpec((1, d), lambda i: (0, 0)),
            pl.BlockSpec((d, dout), lambda i: (0, 0)),
            pl.BlockSpec((1, dout), lambda i: (0, 0)),
        ],
        out_specs=pl.BlockSpec((_ROWS, dout), lambda i: (i, 0)),
        out_shape=jax.ShapeDtypeStruct((n, dout), jnp.float32),
    )(p0, p1, z, deg0, deg1, b, wh, bh)


def kernel(x, edge_index, W1, b1, W2, b2, Wh, bh):
    n, d = x.shape
    src = edge_index[0].astype(jnp.int32)
    dst = edge_index[1].astype(jnp.int32)

    deg2 = _sc_degree(dst, n)
    deg_a, deg_b = deg2[0], deg2[1]

    z1 = _tc_pre(x, deg_a, deg_b, W1)
    pp = _sc_propagate(z1, src, dst)
    z2 = _tc_mid(pp[0], pp[1], z1, deg_a, deg_b, b1.reshape(1, -1), W2)
    qq = _sc_propagate(z2, src, dst)
    out = _tc_post(qq[0], qq[1], z2, deg_a, deg_b, b2.reshape(1, -1),
                   Wh, bh.reshape(1, -1))
    return out


# speedup vs baseline: 12.6483x; 12.6483x over previous
import functools

import jax
import jax.numpy as jnp
from jax import lax
from jax.experimental import pallas as pl
from jax.experimental.pallas import tpu as pltpu
from jax.experimental.pallas import tpu_sc as plsc

_NC = 2
_NS = 16
_CHUNK = 80


def _pad_rows(n):
    g = _NS * 8
    return ((n + g - 1) // g) * g


def _sc_degree(dst_i32, n_nodes):
    e = dst_i32.shape[0]
    nw = _NC * _NS
    assert e % (nw * _CHUNK) == 0, e
    per_tile = e // nw
    chunks = per_tile // _CHUNK
    n_pad = _pad_rows(n_nodes)
    rpt = n_pad // _NS

    mesh = plsc.VectorSubcoreMesh(core_axis_name="c", subcore_axis_name="s")

    @functools.partial(
        pl.kernel,
        out_type=jax.ShapeDtypeStruct((_NC * n_pad,), jnp.float32),
        mesh=mesh,
        scratch_types=[
            pltpu.VMEM((_CHUNK,), jnp.int32),
            pltpu.VMEM((_CHUNK,), jnp.float32),
            pltpu.VMEM((rpt,), jnp.float32),
            pltpu.VMEM_SHARED((n_pad,), jnp.float32),
        ],
    )
    def deg_kernel(dst_hbm, ones_h, zeros_h, out_hbm, didx_v, ones_v,
                   stage_v, acc_sh):
        cid = lax.axis_index("c")
        sid = lax.axis_index("s")
        pltpu.sync_copy(zeros_h.at[pl.ds(sid * rpt, rpt)], stage_v)
        pltpu.sync_copy(stage_v, acc_sh.at[pl.ds(sid * rpt, rpt)])
        pltpu.sync_copy(ones_h, ones_v)
        plsc.subcore_barrier()
        base0 = (cid * _NS + sid) * per_tile

        def body(j, carry):
            pltpu.sync_copy(dst_hbm.at[pl.ds(base0 + j * _CHUNK, _CHUNK)],
                            didx_v)
            pltpu.sync_copy(ones_v, acc_sh.at[didx_v], add=True)
            return carry

        lax.fori_loop(0, chunks, body, 0)
        plsc.subcore_barrier()
        pltpu.sync_copy(acc_sh.at[pl.ds(sid * rpt, rpt)], stage_v)
        pltpu.sync_copy(stage_v,
                        out_hbm.at[pl.ds(cid * n_pad + sid * rpt, rpt)])

    ones = jnp.ones((_CHUNK,), jnp.float32)
    zeros = jnp.zeros((n_pad,), jnp.float32)
    return deg_kernel(dst_i32, ones, zeros)


def _sc_propagate(table, src_i32, dst_i32):
    n_nodes, d = table.shape
    e = src_i32.shape[0]
    nw = _NC * _NS
    assert e % (nw * _CHUNK) == 0, e
    per_tile = e // nw
    chunks = per_tile // _CHUNK
    n_pad = _pad_rows(n_nodes)
    rpt = n_pad // _NS

    mesh = plsc.VectorSubcoreMesh(core_axis_name="c", subcore_axis_name="s")

    @functools.partial(
        pl.kernel,
        out_type=jax.ShapeDtypeStruct((_NC, n_pad, d), jnp.float32),
        mesh=mesh,
        scratch_types=[
            pltpu.VMEM((_CHUNK,), jnp.int32),
            pltpu.VMEM((_CHUNK,), jnp.int32),
            pltpu.VMEM((_CHUNK, d), jnp.float32),
            pltpu.VMEM_SHARED((n_pad, d), jnp.float32),
            pltpu.SemaphoreType.DMA,
        ],
    )
    def prop_kernel(table_hbm, src_hbm, dst_hbm, zeros_h, out_hbm,
                    sidx_v, didx_v, rows_v, acc_sh, sem):
        cid = lax.axis_index("c")
        sid = lax.axis_index("s")
        pltpu.sync_copy(zeros_h.at[pl.ds(sid * rpt, rpt)],
                        acc_sh.at[pl.ds(sid * rpt, rpt)])
        plsc.subcore_barrier()
        base0 = (cid * _NS + sid) * per_tile

        def body(j, carry):
            base = base0 + j * _CHUNK
            pltpu.sync_copy(src_hbm.at[pl.ds(base, _CHUNK)], sidx_v)
            pltpu.sync_copy(dst_hbm.at[pl.ds(base, _CHUNK)], didx_v)
            pltpu.async_copy(table_hbm.at[sidx_v], rows_v, sem).wait()
            pltpu.sync_copy(rows_v, acc_sh.at[didx_v], add=True)
            return carry

        lax.fori_loop(0, chunks, body, 0)
        plsc.subcore_barrier()
        pltpu.sync_copy(acc_sh.at[pl.ds(sid * rpt, rpt)],
                        out_hbm.at[cid, pl.ds(sid * rpt, rpt)])

    zeros = jnp.zeros((n_pad, d), jnp.float32)
    return prop_kernel(table, src_i32, dst_i32, zeros)


_ROWS = 1000


def _tc_pre(x, deg0, deg1, w):
    n, d = x.shape
    dout = w.shape[1]
    assert n % _ROWS == 0

    def body(x_ref, d0_ref, d1_ref, w_ref, o_ref):
        dinv = lax.rsqrt(d0_ref[...] + d1_ref[...] + 1.0)
        o_ref[...] = jnp.dot(x_ref[...] * dinv, w_ref[...],
                             preferred_element_type=jnp.float32)

    return pl.pallas_call(
        body,
        grid=(n // _ROWS,),
        in_specs=[
            pl.BlockSpec((_ROWS, d), lambda i: (i, 0)),
            pl.BlockSpec((_ROWS, 1), lambda i: (i, 0)),
            pl.BlockSpec((_ROWS, 1), lambda i: (i, 0)),
            pl.BlockSpec((d, dout), lambda i: (0, 0)),
        ],
        out_specs=pl.BlockSpec((_ROWS, dout), lambda i: (i, 0)),
        out_shape=jax.ShapeDtypeStruct((n, dout), jnp.float32),
    )(x, deg0, deg1, w)


def _tc_mid(p0, p1, z, deg0, deg1, b, w):
    n, d = z.shape
    dout = w.shape[1]

    def body(p0_ref, p1_ref, z_ref, d0_ref, d1_ref, b_ref, w_ref, o_ref):
        dinv = lax.rsqrt(d0_ref[...] + d1_ref[...] + 1.0)
        pre = (p0_ref[...] + p1_ref[...] + z_ref[...]) * dinv + b_ref[...]
        h = jnp.maximum(pre, 0.0) * dinv
        o_ref[...] = jnp.dot(h, w_ref[...], preferred_element_type=jnp.float32)

    return pl.pallas_call(
        body,
        grid=(n // _ROWS,),
        in_specs=[
            pl.BlockSpec((_ROWS, d), lambda i: (i, 0)),
            pl.BlockSpec((_ROWS, d), lambda i: (i, 0)),
            pl.BlockSpec((_ROWS, d), lambda i: (i, 0)),
            pl.BlockSpec((_ROWS, 1), lambda i: (i, 0)),
            pl.BlockSpec((_ROWS, 1), lambda i: (i, 0)),
            pl.BlockSpec((1, d), lambda i: (0, 0)),
            pl.BlockSpec((d, dout), lambda i: (0, 0)),
        ],
        out_specs=pl.BlockSpec((_ROWS, dout), lambda i: (i, 0)),
        out_shape=jax.ShapeDtypeStruct((n, dout), jnp.float32),
    )(p0, p1, z, deg0, deg1, b, w)


def _tc_post(p0, p1, z, deg0, deg1, b, wh, bh):
    n, d = z.shape
    dout = wh.shape[1]

    def body(p0_ref, p1_ref, z_ref, d0_ref, d1_ref, b_ref, wh_ref, bh_ref,
             o_ref):
        dinv = lax.rsqrt(d0_ref[...] + d1_ref[...] + 1.0)
        pre = (p0_ref[...] + p1_ref[...] + z_ref[...]) * dinv + b_ref[...]
        h = jnp.maximum(pre, 0.0)
        o_ref[...] = jnp.dot(h, wh_ref[...],
                             preferred_element_type=jnp.float32) + bh_ref[...]

    return pl.pallas_call(
        body,
        grid=(n // _ROWS,),
        in_specs=[
            pl.BlockSpec((_ROWS, d), lambda i: (i, 0)),
            pl.BlockSpec((_ROWS, d), lambda i: (i, 0)),
            pl.BlockSpec((_ROWS, d), lambda i: (i, 0)),
            pl.BlockSpec((_ROWS, 1), lambda i: (i, 0)),
            pl.BlockSpec((_ROWS, 1), lambda i: (i, 0)),
            pl.BlockSpec((1, d), lambda i: (0, 0)),
            pl.BlockSpec((d, dout), lambda i: (0, 0)),
            pl.BlockSpec((1, dout), lambda i: (0, 0)),
        ],
        out_specs=pl.BlockSpec((_ROWS, dout), lambda i: (i, 0)),
        out_shape=jax.ShapeDtypeStruct((n, dout), jnp.float32),
    )(p0, p1, z, deg0, deg1, b, wh, bh)


def kernel(x, edge_index, W1, b1, W2, b2, Wh, bh):
    n, d = x.shape
    src = edge_index[0].astype(jnp.int32)
    dst = edge_index[1].astype(jnp.int32)

    n_pad = _pad_rows(n)
    deg2 = _sc_degree(dst, n).reshape(_NC, n_pad, 1)
    deg_a, deg_b = deg2[0], deg2[1]

    z1 = _tc_pre(x, deg_a, deg_b, W1)
    pp = _sc_propagate(z1, src, dst)
    z2 = _tc_mid(pp[0], pp[1], z1, deg_a, deg_b, b1.reshape(1, -1), W2)
    qq = _sc_propagate(z2, src, dst)
    out = _tc_post(qq[0], qq[1], z2, deg_a, deg_b, b2.reshape(1, -1),
                   Wh, bh.reshape(1, -1))
    return out
